# baseline (speedup 1.0000x reference)
import functools

import jax
import jax.numpy as jnp
from jax import lax
from jax.experimental import pallas as pl
from jax.experimental.pallas import tpu as pltpu
from jax.experimental.pallas import tpu_sc as plsc

FIELD_NUM = 26
LATENT_DIM = 16
EPSILON = 1e-3

NUM_CORES = 2
NUM_SUBCORES = 16
NUM_WORKERS = NUM_CORES * NUM_SUBCORES

BLK = 128
CH_BLOCKS = 4
CH = BLK * CH_BLOCKS

KT = 8


def _mesh():
    return plsc.VectorSubcoreMesh(core_axis_name="c", subcore_axis_name="s")


def _params():
    return pltpu.CompilerParams(
        use_tc_tiling_on_sc=False, needs_layout_passes=False
    )


@functools.lru_cache(maxsize=None)
def _build_relayout(feature):
    n_tiles = feature // 128
    n_chunks = -(-n_tiles // KT)
    per_w = -(-n_chunks // NUM_WORKERS)
    chunk_words = KT * 1024
    mesh = _mesh()

    @functools.partial(
        pl.kernel,
        mesh=mesh,
        out_type=jax.ShapeDtypeStruct((feature * LATENT_DIM,), jnp.float32),
        compiler_params=_params(),
        scratch_types=[
            pltpu.VMEM((2 * chunk_words,), jnp.float32),
            pltpu.VMEM((2 * chunk_words,), jnp.float32),
            pltpu.SemaphoreType.DMA,
        ],
    )
    def k(t4_hbm, out_hbm, ibuf, obuf, sem):
        wid = lax.axis_index("s") * NUM_CORES + lax.axis_index("c")
        iota = lax.iota(jnp.int32, 16)
        gvec = (
            lax.shift_left(lax.shift_right_logical(iota, 3), 13)
            + lax.shift_left(iota & 7, 7)
        )

        def body(cc, carry):
            g = wid + cc * NUM_WORKERS
            rt0 = lax.min(g * KT, n_tiles - KT)
            for dt in range(2):
                pltpu.sync_copy(
                    t4_hbm.at[dt, pl.ds(rt0 * 1024, chunk_words)],
                    ibuf.at[pl.ds(dt * chunk_words, chunk_words)],
                )

            def kloop(kk, c2):
                base = gvec + kk * 1024
                obase = kk * 2048
                for c in range(128):
                    vals = plsc.load_gather(ibuf, [base + c])
                    obuf[pl.ds(obase + c * 16, 16)] = vals
                return c2

            lax.fori_loop(0, KT, kloop, 0)
            pltpu.sync_copy(
                obuf, out_hbm.at[pl.ds(rt0 * 2048, 2 * chunk_words)]
            )
            return carry

        lax.fori_loop(0, per_w, body, 0)

    return k


@functools.lru_cache(maxsize=None)
def _build_gather(batch, feature):
    n_rows = batch * FIELD_NUM
    per_w = n_rows // NUM_WORKERS
    n_chunks = per_w // CH
    assert per_w % CH == 0 and batch % BLK == 0
    out_rows = n_rows * LATENT_DIM // 128
    bt_per_f = batch // BLK
    assert batch & (batch - 1) == 0
    bshift = batch.bit_length() - 1
    mesh = _mesh()

    @functools.partial(
        pl.kernel,
        mesh=mesh,
        out_type=jax.ShapeDtypeStruct((out_rows, 128), jnp.float32),
        compiler_params=_params(),
        scratch_types=[
            pltpu.VMEM((CH,), jnp.int32),
            pltpu.VMEM((CH, LATENT_DIM), jnp.float32),
            pltpu.VMEM((2, CH_BLOCKS * 8, 128), jnp.float32),
            pltpu.VMEM((FIELD_NUM * LATENT_DIM,), jnp.float32),
            pltpu.SemaphoreType.DMA,
        ],
    )
    def k(idxf_hbm, arch_hbm, table_hbm, out_hbm,
          idx_v, rows_v, obuf, arch_v, sem):
        wid = lax.axis_index("s") * NUM_CORES + lax.axis_index("c")
        p0w = wid * per_w
        pltpu.sync_copy(arch_hbm, arch_v)
        iota = lax.iota(jnp.int32, 16)
        dvecs = [jnp.full((16,), d, dtype=jnp.int32) for d in range(16)]

        def chunk_body(cc, carry):
            p0 = p0w + cc * CH
            f = lax.shift_right_logical(p0, bshift)
            bt0 = lax.shift_right_logical(p0 & (batch - 1), 7)
            pltpu.sync_copy(idxf_hbm.at[pl.ds(p0, CH)], idx_v)
            cps = [
                pltpu.async_copy(
                    table_hbm.at[idx_v.at[pl.ds(j * BLK, BLK)]],
                    rows_v.at[pl.ds(j * BLK, BLK)],
                    sem,
                )
                for j in range(CH_BLOCKS)
            ]
            for cp in cps:
                cp.wait()
            a = arch_v[pl.ds(f * LATENT_DIM, LATENT_DIM)]
            a2 = a * a
            g = a2 / (a2 + EPSILON)

            def kloop(kb, c2):
                for dt in range(2):
                    for s in range(8):
                        d = dt * 8 + s
                        for cg in range(8):
                            rvec = kb * BLK + cg * 16 + iota
                            vals = plsc.load_gather(rows_v, [rvec, dvecs[d]])
                            obuf[dt, kb * 8 + s, pl.ds(cg * 16, 16)] = (
                                vals * g
                            )
                return c2

            lax.fori_loop(0, CH_BLOCKS, kloop, 0)
            row0 = (f * 2) * (bt_per_f * 8) + bt0 * 8
            for dt in range(2):
                pltpu.sync_copy(
                    obuf.at[dt],
                    out_hbm.at[pl.ds(row0 + dt * bt_per_f * 8, CH_BLOCKS * 8)],
                )
            return carry

        lax.fori_loop(0, n_chunks, chunk_body, 0)

    return k


def kernel(x, arch, embedding):
    batch, fields = x.shape
    feature = embedding.shape[0]
    idx_f = x.T.reshape(-1)
    arch16 = jnp.repeat(arch, LATENT_DIM)
    t4 = (
        embedding.T.reshape(2, 8, feature // 128, 128)
        .transpose(0, 2, 1, 3)
        .reshape(2, feature * 8)
    )
    table_rm = _build_relayout(feature)(t4).reshape(feature, LATENT_DIM)
    out_k = _build_gather(batch, feature)(idx_f, arch16, table_rm)
    return (
        out_k.reshape(fields, 2, batch // 128, 8, 128)
        .transpose(2, 4, 0, 1, 3)
        .reshape(batch, fields, LATENT_DIM)
    )


# speedup vs baseline: 1.1805x; 1.1805x over previous
import functools

import jax
import jax.numpy as jnp
from jax import lax
from jax.experimental import pallas as pl
from jax.experimental.pallas import tpu as pltpu
from jax.experimental.pallas import tpu_sc as plsc

FIELD_NUM = 26
LATENT_DIM = 16
EPSILON = 1e-3

NUM_CORES = 2
NUM_SUBCORES = 16
NUM_WORKERS = NUM_CORES * NUM_SUBCORES

BLK = 128
CH_BLOCKS = 4
CH = BLK * CH_BLOCKS

KT = 8


def _mesh():
    return plsc.VectorSubcoreMesh(core_axis_name="c", subcore_axis_name="s")


def _params():
    return pltpu.CompilerParams(
        use_tc_tiling_on_sc=False, needs_layout_passes=False
    )


@functools.lru_cache(maxsize=None)
def _build_relayout(feature):
    n_tiles = feature // 128
    n_chunks = -(-n_tiles // KT)
    per_w = -(-n_chunks // NUM_WORKERS)
    per_w += per_w % 2
    chunk_words = KT * 1024
    mesh = _mesh()

    @functools.partial(
        pl.kernel,
        mesh=mesh,
        out_type=jax.ShapeDtypeStruct((feature * LATENT_DIM,), jnp.float32),
        compiler_params=_params(),
        scratch_types=[
            pltpu.VMEM((2, 2 * chunk_words), jnp.float32),
            pltpu.VMEM((2, 2 * chunk_words), jnp.float32),
            pltpu.SemaphoreType.DMA,
            pltpu.SemaphoreType.DMA,
            pltpu.SemaphoreType.DMA,
            pltpu.SemaphoreType.DMA,
        ],
    )
    def k(t4_hbm, out_hbm, ibuf, obuf, isem0, isem1, osem0, osem1):
        isems = (isem0, isem1)
        osems = (osem0, osem1)
        wid = lax.axis_index("s") * NUM_CORES + lax.axis_index("c")
        iota = lax.iota(jnp.int32, 16)
        gvec = (
            lax.shift_left(lax.shift_right_logical(iota, 3), 13)
            + lax.shift_left(iota & 7, 7)
        )

        def rt0_of(cc):
            g = wid + cc * NUM_WORKERS
            return lax.min(g * KT, n_tiles - KT)

        def in_copies(cc, b):
            rt0 = rt0_of(cc)
            return [
                pltpu.make_async_copy(
                    t4_hbm.at[dt, pl.ds(rt0 * 1024, chunk_words)],
                    ibuf.at[b, pl.ds(dt * chunk_words, chunk_words)],
                    isems[b],
                )
                for dt in range(2)
            ]

        def out_copy(cc, b):
            rt0 = rt0_of(cc)
            return pltpu.make_async_copy(
                obuf.at[b],
                out_hbm.at[pl.ds(rt0 * 2048, 2 * chunk_words)],
                osems[b],
            )

        for cp in in_copies(0, 0):
            cp.start()
        for cp in in_copies(1, 1):
            cp.start()

        def body(i, carry):
            for b in range(2):
                cc = 2 * i + b
                for cp in in_copies(cc, b):
                    cp.wait()

                @pl.when(cc >= 2)
                def _():
                    out_copy(cc - 2, b).wait()

                def kloop(kk, c2):
                    base = gvec + kk * 1024
                    obase = kk * 2048
                    for c in range(128):
                        vals = plsc.load_gather(ibuf.at[b], [base + c])
                        obuf[b, pl.ds(obase + c * 16, 16)] = vals
                    return c2

                lax.fori_loop(0, KT, kloop, 0)
                if True:
                    @pl.when(cc + 2 < per_w)
                    def _():
                        for cp in in_copies(cc + 2, b):
                            cp.start()
                out_copy(cc, b).start()
            return carry

        lax.fori_loop(0, per_w // 2, body, 0)
        out_copy(per_w - 2, 0).wait()
        out_copy(per_w - 1, 1).wait()

    return k


@functools.lru_cache(maxsize=None)
def _build_gather(batch, feature):
    n_rows = batch * FIELD_NUM
    per_w = n_rows // NUM_WORKERS
    n_chunks = per_w // CH
    assert per_w % CH == 0 and batch % BLK == 0 and n_chunks % 2 == 0
    out_rows = n_rows * LATENT_DIM // 128
    bt_per_f = batch // BLK
    assert batch & (batch - 1) == 0
    bshift = batch.bit_length() - 1
    mesh = _mesh()

    @functools.partial(
        pl.kernel,
        mesh=mesh,
        out_type=jax.ShapeDtypeStruct((out_rows, 128), jnp.float32),
        compiler_params=_params(),
        scratch_types=[
            pltpu.VMEM((2, CH), jnp.int32),
            pltpu.VMEM((2, CH, LATENT_DIM), jnp.float32),
            pltpu.VMEM((2, 2, CH_BLOCKS * 8, 128), jnp.float32),
            pltpu.VMEM((FIELD_NUM * LATENT_DIM,), jnp.float32),
            pltpu.SemaphoreType.DMA,
            pltpu.SemaphoreType.DMA,
            pltpu.SemaphoreType.DMA,
            pltpu.SemaphoreType.DMA,
        ],
    )
    def k(idxf_hbm, arch_hbm, table_hbm, out_hbm,
          idx_v, rows_v, obuf, arch_v, gsem0, gsem1, osem0, osem1):
        gsems = (gsem0, gsem1)
        osems = (osem0, osem1)
        wid = lax.axis_index("s") * NUM_CORES + lax.axis_index("c")
        p0w = wid * per_w
        pltpu.sync_copy(arch_hbm, arch_v)
        iota = lax.iota(jnp.int32, 16)
        dvecs = [jnp.full((16,), d, dtype=jnp.int32) for d in range(16)]

        def gather_copies(b):
            return [
                pltpu.make_async_copy(
                    table_hbm.at[idx_v.at[b, pl.ds(j * BLK, BLK)]],
                    rows_v.at[b, pl.ds(j * BLK, BLK)],
                    gsems[b],
                )
                for j in range(CH_BLOCKS)
            ]

        def fire_chunk(cc, b):
            p0 = p0w + cc * CH
            pltpu.sync_copy(idxf_hbm.at[pl.ds(p0, CH)], idx_v.at[b])
            for cp in gather_copies(b):
                cp.start()

        def out_copies(cc, b):
            p0 = p0w + cc * CH
            f = lax.shift_right_logical(p0, bshift)
            bt0 = lax.shift_right_logical(p0 & (batch - 1), 7)
            row0 = (f * 2) * (bt_per_f * 8) + bt0 * 8
            return [
                pltpu.make_async_copy(
                    obuf.at[b, dt],
                    out_hbm.at[
                        pl.ds(row0 + dt * bt_per_f * 8, CH_BLOCKS * 8)
                    ],
                    osems[b],
                )
                for dt in range(2)
            ]

        fire_chunk(0, 0)

        def body(i, carry):
            for b in range(2):
                cc = 2 * i + b

                @pl.when(cc + 1 < n_chunks)
                def _():
                    fire_chunk(cc + 1, 1 - b)

                for cp in gather_copies(b):
                    cp.wait()

                @pl.when(cc >= 2)
                def _():
                    for cp in out_copies(cc - 2, b):
                        cp.wait()

                p0 = p0w + cc * CH
                f = lax.shift_right_logical(p0, bshift)
                a = arch_v[pl.ds(f * LATENT_DIM, LATENT_DIM)]
                a2 = a * a
                g = a2 / (a2 + EPSILON)

                def kloop(kb, c2):
                    for dt in range(2):
                        for s in range(8):
                            d = dt * 8 + s
                            for cg in range(8):
                                rvec = kb * BLK + cg * 16 + iota
                                vals = plsc.load_gather(
                                    rows_v.at[b], [rvec, dvecs[d]]
                                )
                                obuf[b, dt, kb * 8 + s,
                                     pl.ds(cg * 16, 16)] = vals * g
                    return c2

                lax.fori_loop(0, CH_BLOCKS, kloop, 0)
                for cp in out_copies(cc, b):
                    cp.start()
            return carry

        lax.fori_loop(0, n_chunks // 2, body, 0)
        for cp in out_copies(n_chunks - 2, 0):
            cp.wait()
        for cp in out_copies(n_chunks - 1, 1):
            cp.wait()

    return k


def kernel(x, arch, embedding):
    batch, fields = x.shape
    feature = embedding.shape[0]
    idx_f = x.T.reshape(-1)
    arch16 = jnp.repeat(arch, LATENT_DIM)
    t4 = (
        embedding.T.reshape(2, 8, feature // 128, 128)
        .transpose(0, 2, 1, 3)
        .reshape(2, feature * 8)
    )
    table_rm = _build_relayout(feature)(t4).reshape(feature, LATENT_DIM)
    out_k = _build_gather(batch, feature)(idx_f, arch16, table_rm)
    return (
        out_k.reshape(fields, 2, batch // 128, 8, 128)
        .transpose(2, 4, 0, 1, 3)
        .reshape(batch, fields, LATENT_DIM)
    )


# speedup vs baseline: 2.3046x; 1.9522x over previous
import functools

import jax
import jax.numpy as jnp
from jax import lax
from jax.experimental import pallas as pl
from jax.experimental.pallas import tpu as pltpu
from jax.experimental.pallas import tpu_sc as plsc

FIELD_NUM = 26
LATENT_DIM = 16
EPSILON = 1e-3

NUM_CORES = 2
NUM_SUBCORES = 16
NUM_WORKERS = NUM_CORES * NUM_SUBCORES

BLK = 128
CH_BLOCKS = 4
CH = BLK * CH_BLOCKS

KT = 8


def _mesh():
    return plsc.VectorSubcoreMesh(core_axis_name="c", subcore_axis_name="s")


def _params():
    return pltpu.CompilerParams(
        use_tc_tiling_on_sc=False, needs_layout_passes=False
    )


@functools.lru_cache(maxsize=None)
def _build_relayout(feature):
    n_tiles = feature // 128
    n_chunks = -(-n_tiles // KT)
    per_w = -(-n_chunks // NUM_WORKERS)
    per_w += per_w % 2
    chunk_words = KT * 1024
    mesh = _mesh()

    @functools.partial(
        pl.kernel,
        mesh=mesh,
        out_type=jax.ShapeDtypeStruct((feature * LATENT_DIM,), jnp.float32),
        compiler_params=_params(),
        scratch_types=[
            pltpu.VMEM((2, 2 * chunk_words), jnp.float32),
            pltpu.VMEM((2, 2 * chunk_words), jnp.float32),
            pltpu.SemaphoreType.DMA,
            pltpu.SemaphoreType.DMA,
            pltpu.SemaphoreType.DMA,
            pltpu.SemaphoreType.DMA,
        ],
    )
    def k(t4_hbm, out_hbm, ibuf, obuf, isem0, isem1, osem0, osem1):
        isems = (isem0, isem1)
        osems = (osem0, osem1)
        wid = lax.axis_index("s") * NUM_CORES + lax.axis_index("c")
        iota = lax.iota(jnp.int32, 16)
        gvec = (
            lax.shift_left(lax.shift_right_logical(iota, 3), 13)
            + lax.shift_left(iota & 7, 7)
        )

        def rt0_of(cc):
            g = wid + cc * NUM_WORKERS
            return lax.min(g * KT, n_tiles - KT)

        def in_copies(cc, b):
            rt0 = rt0_of(cc)
            return [
                pltpu.make_async_copy(
                    t4_hbm.at[dt, pl.ds(rt0 * 1024, chunk_words)],
                    ibuf.at[b, pl.ds(dt * chunk_words, chunk_words)],
                    isems[b],
                )
                for dt in range(2)
            ]

        def out_copy(cc, b):
            rt0 = rt0_of(cc)
            return pltpu.make_async_copy(
                obuf.at[b],
                out_hbm.at[pl.ds(rt0 * 2048, 2 * chunk_words)],
                osems[b],
            )

        for cp in in_copies(0, 0):
            cp.start()
        for cp in in_copies(1, 1):
            cp.start()

        def body(i, carry):
            for b in range(2):
                cc = 2 * i + b
                for cp in in_copies(cc, b):
                    cp.wait()

                @pl.when(cc >= 2)
                def _():
                    out_copy(cc - 2, b).wait()

                @plsc.parallel_loop(0, KT * 128, unroll=8)
                def _(r):
                    kk = lax.shift_right_logical(r, 7)
                    c = r & 127
                    vals = plsc.load_gather(
                        ibuf.at[b], [gvec + (kk * 1024 + c)]
                    )
                    obuf[b, pl.ds(r * 16, 16)] = vals
                if True:
                    @pl.when(cc + 2 < per_w)
                    def _():
                        for cp in in_copies(cc + 2, b):
                            cp.start()
                out_copy(cc, b).start()
            return carry

        lax.fori_loop(0, per_w // 2, body, 0)
        out_copy(per_w - 2, 0).wait()
        out_copy(per_w - 1, 1).wait()

    return k


@functools.lru_cache(maxsize=None)
def _build_gather(batch, feature):
    n_rows = batch * FIELD_NUM
    per_w = n_rows // NUM_WORKERS
    n_chunks = per_w // CH
    assert per_w % CH == 0 and batch % BLK == 0 and n_chunks % 2 == 0
    out_rows = n_rows * LATENT_DIM // 128
    bt_per_f = batch // BLK
    assert batch & (batch - 1) == 0
    bshift = batch.bit_length() - 1
    mesh = _mesh()

    @functools.partial(
        pl.kernel,
        mesh=mesh,
        out_type=jax.ShapeDtypeStruct((out_rows, 128), jnp.float32),
        compiler_params=_params(),
        scratch_types=[
            pltpu.VMEM((2, CH), jnp.int32),
            pltpu.VMEM((2, CH, LATENT_DIM), jnp.float32),
            pltpu.VMEM((2, 2, CH_BLOCKS * 8, 128), jnp.float32),
            pltpu.VMEM((FIELD_NUM * LATENT_DIM,), jnp.float32),
            pltpu.SemaphoreType.DMA,
            pltpu.SemaphoreType.DMA,
            pltpu.SemaphoreType.DMA,
            pltpu.SemaphoreType.DMA,
        ],
    )
    def k(idxf_hbm, arch_hbm, table_hbm, out_hbm,
          idx_v, rows_v, obuf, arch_v, gsem0, gsem1, osem0, osem1):
        gsems = (gsem0, gsem1)
        osems = (osem0, osem1)
        wid = lax.axis_index("s") * NUM_CORES + lax.axis_index("c")
        p0w = wid * per_w
        pltpu.sync_copy(arch_hbm, arch_v)
        iota = lax.iota(jnp.int32, 16)
        dvecs = [jnp.full((16,), d, dtype=jnp.int32) for d in range(16)]

        def gather_copies(b):
            return [
                pltpu.make_async_copy(
                    table_hbm.at[idx_v.at[b, pl.ds(j * BLK, BLK)]],
                    rows_v.at[b, pl.ds(j * BLK, BLK)],
                    gsems[b],
                )
                for j in range(CH_BLOCKS)
            ]

        def fire_chunk(cc, b):
            p0 = p0w + cc * CH
            pltpu.sync_copy(idxf_hbm.at[pl.ds(p0, CH)], idx_v.at[b])
            for cp in gather_copies(b):
                cp.start()

        def out_copies(cc, b):
            p0 = p0w + cc * CH
            f = lax.shift_right_logical(p0, bshift)
            bt0 = lax.shift_right_logical(p0 & (batch - 1), 7)
            row0 = (f * 2) * (bt_per_f * 8) + bt0 * 8
            return [
                pltpu.make_async_copy(
                    obuf.at[b, dt],
                    out_hbm.at[
                        pl.ds(row0 + dt * bt_per_f * 8, CH_BLOCKS * 8)
                    ],
                    osems[b],
                )
                for dt in range(2)
            ]

        fire_chunk(0, 0)

        def body(i, carry):
            for b in range(2):
                cc = 2 * i + b

                @pl.when(cc + 1 < n_chunks)
                def _():
                    fire_chunk(cc + 1, 1 - b)

                for cp in gather_copies(b):
                    cp.wait()

                @pl.when(cc >= 2)
                def _():
                    for cp in out_copies(cc - 2, b):
                        cp.wait()

                p0 = p0w + cc * CH
                f = lax.shift_right_logical(p0, bshift)
                a = arch_v[pl.ds(f * LATENT_DIM, LATENT_DIM)]
                a2 = a * a
                g = a2 / (a2 + EPSILON)

                @plsc.parallel_loop(0, CH_BLOCKS * 8, unroll=4)
                def _(it):
                    kb = lax.shift_right_logical(it, 3)
                    cg = it & 7
                    rvec = kb * BLK + cg * 16 + iota
                    for dt in range(2):
                        for s in range(8):
                            d = dt * 8 + s
                            vals = plsc.load_gather(
                                rows_v.at[b], [rvec, dvecs[d]]
                            )
                            obuf[b, dt, kb * 8 + s,
                                 pl.ds(cg * 16, 16)] = vals * g
                for cp in out_copies(cc, b):
                    cp.start()
            return carry

        lax.fori_loop(0, n_chunks // 2, body, 0)
        for cp in out_copies(n_chunks - 2, 0):
            cp.wait()
        for cp in out_copies(n_chunks - 1, 1):
            cp.wait()

    return k


def kernel(x, arch, embedding):
    batch, fields = x.shape
    feature = embedding.shape[0]
    idx_f = x.T.reshape(-1)
    arch16 = jnp.repeat(arch, LATENT_DIM)
    t4 = (
        embedding.T.reshape(2, 8, feature // 128, 128)
        .transpose(0, 2, 1, 3)
        .reshape(2, feature * 8)
    )
    table_rm = _build_relayout(feature)(t4).reshape(feature, LATENT_DIM)
    out_k = _build_gather(batch, feature)(idx_f, arch16, table_rm)
    return (
        out_k.reshape(fields, 2, batch // 128, 8, 128)
        .transpose(2, 4, 0, 1, 3)
        .reshape(batch, fields, LATENT_DIM)
    )


# speedup vs baseline: 2.4346x; 1.0564x over previous
import functools

import jax
import jax.numpy as jnp
from jax import lax
from jax.experimental import pallas as pl
from jax.experimental.pallas import tpu as pltpu
from jax.experimental.pallas import tpu_sc as plsc

FIELD_NUM = 26
LATENT_DIM = 16
EPSILON = 1e-3

NUM_CORES = 2
NUM_SUBCORES = 16
NUM_WORKERS = NUM_CORES * NUM_SUBCORES

BLK = 128
CH_BLOCKS = 4
CH = BLK * CH_BLOCKS

KT = 8


def _mesh():
    return plsc.VectorSubcoreMesh(core_axis_name="c", subcore_axis_name="s")


def _params():
    return pltpu.CompilerParams(
        use_tc_tiling_on_sc=False, needs_layout_passes=False
    )


@functools.lru_cache(maxsize=None)
def _build_relayout(feature):
    n_tiles = feature // 128
    n_chunks = -(-n_tiles // KT)
    per_w = -(-n_chunks // NUM_WORKERS)
    per_w += per_w % 2
    chunk_words = KT * 1024
    mesh = _mesh()

    @functools.partial(
        pl.kernel,
        mesh=mesh,
        out_type=jax.ShapeDtypeStruct((feature * LATENT_DIM,), jnp.float32),
        compiler_params=_params(),
        scratch_types=[
            pltpu.VMEM((2, 2 * chunk_words), jnp.float32),
            pltpu.VMEM((2, 2 * chunk_words), jnp.float32),
            pltpu.SemaphoreType.DMA,
            pltpu.SemaphoreType.DMA,
            pltpu.SemaphoreType.DMA,
            pltpu.SemaphoreType.DMA,
        ],
    )
    def k(t4_hbm, out_hbm, ibuf, obuf, isem0, isem1, osem0, osem1):
        isems = (isem0, isem1)
        osems = (osem0, osem1)
        wid = lax.axis_index("s") * NUM_CORES + lax.axis_index("c")
        iota = lax.iota(jnp.int32, 16)
        gvec = (
            lax.shift_left(lax.shift_right_logical(iota, 3), 13)
            + lax.shift_left(iota & 7, 7)
        )

        def rt0_of(cc):
            g = wid + cc * NUM_WORKERS
            return lax.min(g * KT, n_tiles - KT)

        def in_copies(cc, b):
            rt0 = rt0_of(cc)
            return [
                pltpu.make_async_copy(
                    t4_hbm.at[dt, pl.ds(rt0 * 1024, chunk_words)],
                    ibuf.at[b, pl.ds(dt * chunk_words, chunk_words)],
                    isems[b],
                )
                for dt in range(2)
            ]

        def out_copy(cc, b):
            rt0 = rt0_of(cc)
            return pltpu.make_async_copy(
                obuf.at[b],
                out_hbm.at[pl.ds(rt0 * 2048, 2 * chunk_words)],
                osems[b],
            )

        for cp in in_copies(0, 0):
            cp.start()
        for cp in in_copies(1, 1):
            cp.start()

        def body(i, carry):
            for b in range(2):
                cc = 2 * i + b
                for cp in in_copies(cc, b):
                    cp.wait()

                @pl.when(cc >= 2)
                def _():
                    out_copy(cc - 2, b).wait()

                @plsc.parallel_loop(0, KT * 128, unroll=16)
                def _(r):
                    kk = lax.shift_right_logical(r, 7)
                    c = r & 127
                    vals = plsc.load_gather(
                        ibuf.at[b], [gvec + (kk * 1024 + c)]
                    )
                    obuf[b, pl.ds(r * 16, 16)] = vals
                if True:
                    @pl.when(cc + 2 < per_w)
                    def _():
                        for cp in in_copies(cc + 2, b):
                            cp.start()
                out_copy(cc, b).start()
            return carry

        lax.fori_loop(0, per_w // 2, body, 0)
        out_copy(per_w - 2, 0).wait()
        out_copy(per_w - 1, 1).wait()

    return k


@functools.lru_cache(maxsize=None)
def _build_gather(batch, feature):
    n_rows = batch * FIELD_NUM
    per_w = n_rows // NUM_WORKERS
    n_chunks = per_w // CH
    assert per_w % CH == 0 and batch % BLK == 0 and n_chunks % 2 == 0
    out_rows = n_rows * LATENT_DIM // 128
    bt_per_f = batch // BLK
    assert batch & (batch - 1) == 0
    bshift = batch.bit_length() - 1
    mesh = _mesh()

    @functools.partial(
        pl.kernel,
        mesh=mesh,
        out_type=jax.ShapeDtypeStruct((out_rows, 128), jnp.float32),
        compiler_params=_params(),
        scratch_types=[
            pltpu.VMEM((2, CH), jnp.int32),
            pltpu.VMEM((2, CH, LATENT_DIM), jnp.float32),
            pltpu.VMEM((2, 2, CH_BLOCKS * 8, 128), jnp.float32),
            pltpu.VMEM((FIELD_NUM * LATENT_DIM,), jnp.float32),
            pltpu.SemaphoreType.DMA,
            pltpu.SemaphoreType.DMA,
            pltpu.SemaphoreType.DMA,
            pltpu.SemaphoreType.DMA,
        ],
    )
    def k(idxf_hbm, arch_hbm, table_hbm, out_hbm,
          idx_v, rows_v, obuf, arch_v, gsem0, gsem1, osem0, osem1):
        gsems = (gsem0, gsem1)
        osems = (osem0, osem1)
        wid = lax.axis_index("s") * NUM_CORES + lax.axis_index("c")
        p0w = wid * per_w
        pltpu.sync_copy(arch_hbm, arch_v)
        iota = lax.iota(jnp.int32, 16)
        dvecs = [jnp.full((16,), d, dtype=jnp.int32) for d in range(16)]

        def gather_copies(b):
            return [
                pltpu.make_async_copy(
                    table_hbm.at[idx_v.at[b, pl.ds(j * BLK, BLK)]],
                    rows_v.at[b, pl.ds(j * BLK, BLK)],
                    gsems[b],
                )
                for j in range(CH_BLOCKS)
            ]

        def fire_chunk(cc, b):
            p0 = p0w + cc * CH
            pltpu.sync_copy(idxf_hbm.at[pl.ds(p0, CH)], idx_v.at[b])
            for cp in gather_copies(b):
                cp.start()

        def out_copies(cc, b):
            p0 = p0w + cc * CH
            f = lax.shift_right_logical(p0, bshift)
            bt0 = lax.shift_right_logical(p0 & (batch - 1), 7)
            row0 = (f * 2) * (bt_per_f * 8) + bt0 * 8
            return [
                pltpu.make_async_copy(
                    obuf.at[b, dt],
                    out_hbm.at[
                        pl.ds(row0 + dt * bt_per_f * 8, CH_BLOCKS * 8)
                    ],
                    osems[b],
                )
                for dt in range(2)
            ]

        fire_chunk(0, 0)

        def body(i, carry):
            for b in range(2):
                cc = 2 * i + b

                @pl.when(cc + 1 < n_chunks)
                def _():
                    fire_chunk(cc + 1, 1 - b)

                for cp in gather_copies(b):
                    cp.wait()

                @pl.when(cc >= 2)
                def _():
                    for cp in out_copies(cc - 2, b):
                        cp.wait()

                p0 = p0w + cc * CH
                f = lax.shift_right_logical(p0, bshift)
                a = arch_v[pl.ds(f * LATENT_DIM, LATENT_DIM)]
                a2 = a * a
                g = a2 / (a2 + EPSILON)

                @plsc.parallel_loop(0, CH_BLOCKS * 8, unroll=8)
                def _(it):
                    kb = lax.shift_right_logical(it, 3)
                    cg = it & 7
                    rvec = kb * BLK + cg * 16 + iota
                    for dt in range(2):
                        for s in range(8):
                            d = dt * 8 + s
                            vals = plsc.load_gather(
                                rows_v.at[b], [rvec, dvecs[d]]
                            )
                            obuf[b, dt, kb * 8 + s,
                                 pl.ds(cg * 16, 16)] = vals * g
                for cp in out_copies(cc, b):
                    cp.start()
            return carry

        lax.fori_loop(0, n_chunks // 2, body, 0)
        for cp in out_copies(n_chunks - 2, 0):
            cp.wait()
        for cp in out_copies(n_chunks - 1, 1):
            cp.wait()

    return k


def kernel(x, arch, embedding):
    batch, fields = x.shape
    feature = embedding.shape[0]
    idx_f = x.T.reshape(-1)
    arch16 = jnp.repeat(arch, LATENT_DIM)
    t4 = (
        embedding.T.reshape(2, 8, feature // 128, 128)
        .transpose(0, 2, 1, 3)
        .reshape(2, feature * 8)
    )
    table_rm = _build_relayout(feature)(t4).reshape(feature, LATENT_DIM)
    out_k = _build_gather(batch, feature)(idx_f, arch16, table_rm)
    return (
        out_k.reshape(fields, 2, batch // 128, 8, 128)
        .transpose(2, 4, 0, 1, 3)
        .reshape(batch, fields, LATENT_DIM)
    )


# speedup vs baseline: 4.9346x; 2.0268x over previous
import functools

import jax
import jax.numpy as jnp
from jax import lax
from jax.experimental import pallas as pl
from jax.experimental.pallas import tpu as pltpu
from jax.experimental.pallas import tpu_sc as plsc

FIELD_NUM = 26
LATENT_DIM = 16
EPSILON = 1e-3

NUM_CORES = 2
NUM_SUBCORES = 16
NUM_WORKERS = NUM_CORES * NUM_SUBCORES

BLK = 128
CH_BLOCKS = 4
CH = BLK * CH_BLOCKS

KT = 8


def _mesh():
    return plsc.VectorSubcoreMesh(core_axis_name="c", subcore_axis_name="s")


def _params():
    return pltpu.CompilerParams(
        use_tc_tiling_on_sc=False, needs_layout_passes=False
    )


@functools.lru_cache(maxsize=None)
def _build_relayout(feature):
    n_tiles = feature // 128
    n_chunks = -(-n_tiles // KT)
    per_w = -(-n_chunks // NUM_WORKERS)
    per_w += per_w % 2
    chunk_words = KT * 1024
    mesh = _mesh()

    @functools.partial(
        pl.kernel,
        mesh=mesh,
        out_type=jax.ShapeDtypeStruct((feature * LATENT_DIM,), jnp.float32),
        compiler_params=_params(),
        scratch_types=[
            pltpu.VMEM((2, 2, KT * 8, 129), jnp.float32),
            pltpu.VMEM((2, 2 * chunk_words), jnp.float32),
            pltpu.SemaphoreType.DMA,
            pltpu.SemaphoreType.DMA,
            pltpu.SemaphoreType.DMA,
            pltpu.SemaphoreType.DMA,
        ],
    )
    def k(t4_hbm, out_hbm, ibuf, obuf, isem0, isem1, osem0, osem1):
        isems = (isem0, isem1)
        osems = (osem0, osem1)
        wid = lax.axis_index("s") * NUM_CORES + lax.axis_index("c")
        iota = lax.iota(jnp.int32, 16)
        dt_vec = lax.shift_right_logical(iota, 3)
        s_vec = iota & 7

        def rt0_of(cc):
            g = wid + cc * NUM_WORKERS
            return lax.min(g * KT, n_tiles - KT)

        def in_copies(cc, b):
            rt0 = rt0_of(cc)
            return [
                pltpu.make_async_copy(
                    t4_hbm.at[dt, pl.ds(rt0 * 8, KT * 8)],
                    ibuf.at[b, dt, pl.ds(0, KT * 8), pl.ds(0, 128)],
                    isems[b],
                )
                for dt in range(2)
            ]

        def out_copy(cc, b):
            rt0 = rt0_of(cc)
            return pltpu.make_async_copy(
                obuf.at[b],
                out_hbm.at[pl.ds(rt0 * 2048, 2 * chunk_words)],
                osems[b],
            )

        for cp in in_copies(0, 0):
            cp.start()
        for cp in in_copies(1, 1):
            cp.start()

        def body(i, carry):
            for b in range(2):
                cc = 2 * i + b
                for cp in in_copies(cc, b):
                    cp.wait()

                @pl.when(cc >= 2)
                def _():
                    out_copy(cc - 2, b).wait()

                @plsc.parallel_loop(0, KT * 128, unroll=16)
                def _(r):
                    kk = lax.shift_right_logical(r, 7)
                    c = r & 127
                    vals = plsc.load_gather(
                        ibuf.at[b],
                        [dt_vec, s_vec + kk * 8, jnp.full((16,), c, jnp.int32)],
                    )
                    obuf[b, pl.ds(r * 16, 16)] = vals
                if True:
                    @pl.when(cc + 2 < per_w)
                    def _():
                        for cp in in_copies(cc + 2, b):
                            cp.start()
                out_copy(cc, b).start()
            return carry

        lax.fori_loop(0, per_w // 2, body, 0)
        out_copy(per_w - 2, 0).wait()
        out_copy(per_w - 1, 1).wait()

    return k


@functools.lru_cache(maxsize=None)
def _build_gather(batch, feature):
    n_rows = batch * FIELD_NUM
    per_w = n_rows // NUM_WORKERS
    n_chunks = per_w // CH
    assert per_w % CH == 0 and batch % BLK == 0 and n_chunks % 2 == 0
    out_rows = n_rows * LATENT_DIM // 128
    bt_per_f = batch // BLK
    assert batch & (batch - 1) == 0
    bshift = batch.bit_length() - 1
    mesh = _mesh()

    @functools.partial(
        pl.kernel,
        mesh=mesh,
        out_type=jax.ShapeDtypeStruct((out_rows, 128), jnp.float32),
        compiler_params=_params(),
        scratch_types=[
            pltpu.VMEM((2, CH), jnp.int32),
            pltpu.VMEM((2, CH, LATENT_DIM), jnp.float32),
            pltpu.VMEM((CH, LATENT_DIM + 1), jnp.float32),
            pltpu.VMEM((2, 2, CH_BLOCKS * 8, 128), jnp.float32),
            pltpu.VMEM((FIELD_NUM * LATENT_DIM,), jnp.float32),
            pltpu.SemaphoreType.DMA,
            pltpu.SemaphoreType.DMA,
            pltpu.SemaphoreType.DMA,
            pltpu.SemaphoreType.DMA,
        ],
    )
    def k(idxf_hbm, arch_hbm, table_hbm, out_hbm,
          idx_v, rows_v, pbuf, obuf, arch_v, gsem0, gsem1, osem0, osem1):
        gsems = (gsem0, gsem1)
        osems = (osem0, osem1)
        wid = lax.axis_index("s") * NUM_CORES + lax.axis_index("c")
        p0w = wid * per_w
        pltpu.sync_copy(arch_hbm, arch_v)
        iota = lax.iota(jnp.int32, 16)
        dvecs = [jnp.full((16,), d, dtype=jnp.int32) for d in range(16)]

        def gather_copies(b):
            return [
                pltpu.make_async_copy(
                    table_hbm.at[idx_v.at[b, pl.ds(j * BLK, BLK)]],
                    rows_v.at[b, pl.ds(j * BLK, BLK)],
                    gsems[b],
                )
                for j in range(CH_BLOCKS)
            ]

        def fire_chunk(cc, b):
            p0 = p0w + cc * CH
            pltpu.sync_copy(idxf_hbm.at[pl.ds(p0, CH)], idx_v.at[b])
            for cp in gather_copies(b):
                cp.start()

        def out_copies(cc, b):
            p0 = p0w + cc * CH
            f = lax.shift_right_logical(p0, bshift)
            bt0 = lax.shift_right_logical(p0 & (batch - 1), 7)
            row0 = (f * 2) * (bt_per_f * 8) + bt0 * 8
            return [
                pltpu.make_async_copy(
                    obuf.at[b, dt],
                    out_hbm.at[
                        pl.ds(row0 + dt * bt_per_f * 8, CH_BLOCKS * 8)
                    ],
                    osems[b],
                )
                for dt in range(2)
            ]

        fire_chunk(0, 0)

        def body(i, carry):
            for b in range(2):
                cc = 2 * i + b

                @pl.when(cc + 1 < n_chunks)
                def _():
                    fire_chunk(cc + 1, 1 - b)

                for cp in gather_copies(b):
                    cp.wait()

                @pl.when(cc >= 2)
                def _():
                    for cp in out_copies(cc - 2, b):
                        cp.wait()

                p0 = p0w + cc * CH
                f = lax.shift_right_logical(p0, bshift)
                a = arch_v[pl.ds(f * LATENT_DIM, LATENT_DIM)]
                a2 = a * a
                g = a2 / (a2 + EPSILON)

                @plsc.parallel_loop(0, CH, unroll=16)
                def _(r):
                    pbuf[r, pl.ds(0, LATENT_DIM)] = rows_v[b, r]

                @plsc.parallel_loop(0, CH_BLOCKS * 8, unroll=8)
                def _(it):
                    kb = lax.shift_right_logical(it, 3)
                    cg = it & 7
                    rvec = kb * BLK + cg * 16 + iota
                    for dt in range(2):
                        for s in range(8):
                            d = dt * 8 + s
                            vals = plsc.load_gather(pbuf, [rvec, dvecs[d]])
                            obuf[b, dt, kb * 8 + s,
                                 pl.ds(cg * 16, 16)] = vals * g
                for cp in out_copies(cc, b):
                    cp.start()
            return carry

        lax.fori_loop(0, n_chunks // 2, body, 0)
        for cp in out_copies(n_chunks - 2, 0):
            cp.wait()
        for cp in out_copies(n_chunks - 1, 1):
            cp.wait()

    return k


def kernel(x, arch, embedding):
    batch, fields = x.shape
    feature = embedding.shape[0]
    idx_f = x.T.reshape(-1)
    arch16 = jnp.repeat(arch, LATENT_DIM)
    t4 = (
        embedding.T.reshape(2, 8, feature // 128, 128)
        .transpose(0, 2, 1, 3)
        .reshape(2, feature // 16, 128)
    )
    table_rm = _build_relayout(feature)(t4).reshape(feature, LATENT_DIM)
    out_k = _build_gather(batch, feature)(idx_f, arch16, table_rm)
    return (
        out_k.reshape(fields, 2, batch // 128, 8, 128)
        .transpose(2, 4, 0, 1, 3)
        .reshape(batch, fields, LATENT_DIM)
    )
